# Initial kernel scaffold; baseline (speedup 1.0000x reference)
import jax
import jax.numpy as jnp
from jax.experimental import pallas as pl


def kernel(scores):
    raise NotImplementedError("write your pallas kernel here")


# speedup vs baseline: 2.0304x; 2.0304x over previous
import functools

import jax
import jax.numpy as jnp
from jax import lax
from jax.experimental import pallas as pl
from jax.experimental.pallas import tpu as pltpu
from jax.experimental.pallas import tpu_sc as plsc

ROWS = 128
COLS = 32768
LANES = 16
CHUNKS = COLS // LANES
NC, NS = 2, 16
NW = NC * NS
ROWS_PER_W = ROWS // NW
NEG = -1e30


def _sc_topk_body(scores_hbm, vals_hbm, ids_hbm, row_v, vrow_v, irow_v):
    wid = lax.axis_index("s") * NC + lax.axis_index("c")

    for t in range(3, 8):
        vrow_v[pl.ds(16 * t, 16)] = jnp.full((16,), NEG, jnp.float32)
        irow_v[pl.ds(16 * t, 16)] = jnp.zeros((16,), jnp.int32)

    for r in range(ROWS_PER_W):
        row = wid * ROWS_PER_W + r
        pltpu.sync_copy(scores_hbm.at[row], row_v)

        def scan_step(c, carry):
            m1, m2, m3, i1, i2, i3 = carry
            x = row_v[pl.ds(c * LANES, LANES)]
            t1 = x > m1
            t2 = x > m2
            t3 = x > m3
            n_m3 = jnp.where(t2, m2, jnp.where(t3, x, m3))
            n_i3 = jnp.where(t2, i2, jnp.where(t3, c, i3))
            n_m2 = jnp.where(t1, m1, jnp.where(t2, x, m2))
            n_i2 = jnp.where(t1, i1, jnp.where(t2, c, i2))
            n_m1 = jnp.where(t1, x, m1)
            n_i1 = jnp.where(t1, c, i1)
            return n_m1, n_m2, n_m3, n_i1, n_i2, n_i3

        init = (
            jnp.full((16,), NEG, jnp.float32),
            jnp.full((16,), NEG, jnp.float32),
            jnp.full((16,), NEG, jnp.float32),
            jnp.zeros((16,), jnp.int32),
            jnp.zeros((16,), jnp.int32),
            jnp.zeros((16,), jnp.int32),
        )
        m1, m2, m3, i1, i2, i3 = lax.fori_loop(0, CHUNKS, scan_step, init)

        vrow_v[pl.ds(0, 16)] = m1
        vrow_v[pl.ds(16, 16)] = m2
        vrow_v[pl.ds(32, 16)] = m3
        irow_v[pl.ds(0, 16)] = i1
        irow_v[pl.ds(16, 16)] = i2
        irow_v[pl.ds(32, 16)] = i3
        pltpu.sync_copy(vrow_v, vals_hbm.at[row])
        pltpu.sync_copy(irow_v, ids_hbm.at[row])


def _sc_topk(scores):
    mesh = plsc.VectorSubcoreMesh(
        core_axis_name="c", subcore_axis_name="s", num_cores=NC, num_subcores=NS
    )
    fn = pl.kernel(
        _sc_topk_body,
        out_type=[
            jax.ShapeDtypeStruct((ROWS, 128), jnp.float32),
            jax.ShapeDtypeStruct((ROWS, 128), jnp.int32),
        ],
        mesh=mesh,
        scratch_types=[
            pltpu.VMEM((COLS,), jnp.float32),
            pltpu.VMEM((128,), jnp.float32),
            pltpu.VMEM((128,), jnp.int32),
        ],
    )
    return fn(scores)


def _tc_write_body(vref, iref, out_ref):
    v = vref[...]
    cid = iref[...]
    lane = lax.broadcasted_iota(jnp.int32, v.shape, 1) & (LANES - 1)
    colidx = cid * LANES + lane

    vals, idxs = [], []
    vv = v
    for _ in range(3):
        m = jnp.max(vv, axis=1, keepdims=True)
        sel = vv == m
        ik = jnp.min(
            jnp.where(sel, colidx, jnp.int32(1 << 30)), axis=1, keepdims=True
        )
        vals.append(m)
        idxs.append(ik)
        vv = jnp.where(colidx == ik, NEG, vv)

    e1 = jnp.exp(vals[0] - vals[0])
    e2 = jnp.exp(vals[1] - vals[0])
    e3 = jnp.exp(vals[2] - vals[0])
    denom = e1 + e2 + e3
    p1, p2, p3 = e1 / denom, e2 / denom, e3 / denom

    cix = lax.broadcasted_iota(jnp.int32, out_ref.shape, 1)
    zero = jnp.float32(0.0)
    out = (
        jnp.where(cix == idxs[0], p1, zero)
        + jnp.where(cix == idxs[1], p2, zero)
        + jnp.where(cix == idxs[2], p3, zero)
    )
    out_ref[...] = out


def _tc_write(cand_v, cand_i):
    rb = 8
    return pl.pallas_call(
        _tc_write_body,
        grid=(ROWS // rb,),
        in_specs=[
            pl.BlockSpec((rb, 128), lambda i: (i, 0)),
            pl.BlockSpec((rb, 128), lambda i: (i, 0)),
        ],
        out_specs=pl.BlockSpec((rb, COLS), lambda i: (i, 0)),
        out_shape=jax.ShapeDtypeStruct((ROWS, COLS), jnp.float32),
    )(cand_v, cand_i)


def kernel(scores):
    cand_v, cand_i = _sc_topk(scores)
    return _tc_write(cand_v, cand_i)
